# Initial kernel scaffold; baseline (speedup 1.0000x reference)
import jax
import jax.numpy as jnp
from jax.experimental import pallas as pl


def kernel(x, edge_index, edge_weight, W0, b0, W1, b1, W2, b2):
    raise NotImplementedError("write your pallas kernel here")


# speedup vs baseline: 8.5193x; 8.5193x over previous
import functools

import jax
import jax.numpy as jnp
from jax import lax
from jax.experimental import pallas as pl
from jax.experimental.pallas import tpu as pltpu
from jax.experimental.pallas import tpu_sc as plsc

f32 = jnp.float32

NC = 2
NS = 16
NW = NC * NS
DEGW = 16


def _make_agg(n_nodes, d_feat, n_edges, K, with_deg):
    n_chunks = n_edges // K
    cpt = n_chunks // NW
    assert cpt % 2 == 1 and cpt * NW == n_chunks
    nzt = 10
    rpt = n_nodes // nzt
    assert rpt % 8 == 0 and rpt * nzt == n_nodes
    mesh = plsc.VectorSubcoreMesh(
        core_axis_name="c", subcore_axis_name="s",
        num_cores=NC, num_subcores=NS)

    out_type = [jax.ShapeDtypeStruct((NC, n_nodes, d_feat), f32)]
    scratch = [
        pltpu.VMEM_SHARED((n_nodes, d_feat), f32),
        pltpu.VMEM((cpt, K), jnp.int32),
        pltpu.VMEM((cpt, K), jnp.int32),
        pltpu.VMEM((cpt, K), f32),
        pltpu.VMEM((K, d_feat), f32),
        pltpu.VMEM((K, d_feat), f32),
        pltpu.SemaphoreType.DMA,
        pltpu.SemaphoreType.DMA,
    ]
    if with_deg:
        out_type.append(jax.ShapeDtypeStruct((NC, n_nodes, DEGW), f32))
        scratch += [
            pltpu.VMEM_SHARED((n_nodes, DEGW), f32),
            pltpu.VMEM((K, DEGW), f32),
        ]

    def body(h_hbm, src_hbm, dst_hbm, w_hbm, z_hbm, zd_hbm, *rest):
        if with_deg:
            (out_hbm, outdeg_hbm, acc, srcb, dstb, wb, rows0, rows1,
             sem0, sem1, dacc, wrow) = rest
        else:
            (out_hbm, acc, srcb, dstb, wb, rows0, rows1, sem0, sem1) = rest
        cid = lax.axis_index("c")
        sid = lax.axis_index("s")
        tid = cid * NS + sid
        r0 = sid * rpt

        @pl.when(sid < nzt)
        def _():
            pltpu.sync_copy(z_hbm, acc.at[pl.ds(r0, rpt)])
            if with_deg:
                pltpu.sync_copy(zd_hbm, dacc.at[pl.ds(r0, rpt)])

        pltpu.sync_copy(src_hbm.at[tid], srcb)
        pltpu.sync_copy(dst_hbm.at[tid], dstb)
        pltpu.sync_copy(w_hbm.at[tid], wb)

        plsc.subcore_barrier()

        def start_gather(c, rows, sem):
            pltpu.async_copy(h_hbm.at[srcb.at[c]], rows, sem)

        def wait_gather(c, rows, sem):
            pltpu.make_async_copy(h_hbm.at[srcb.at[c]], rows, sem).wait()

        def process(c, rows):
            @pl.loop(0, K // 16)
            def _(g):
                wvec = wb[c, pl.ds(g * 16, 16)]
                for eo in range(16):
                    e = g * 16 + eo
                    wv = jnp.full((16,), wvec[eo], f32)
                    if with_deg:
                        wrow[e, :] = wv
                    for j in range(d_feat // 16):
                        sl = pl.ds(j * 16, 16)
                        rows[e, sl] = rows[e, sl] * wv
            if with_deg:
                pltpu.sync_copy(wrow, dacc.at[dstb.at[c]], add=True)
            pltpu.sync_copy(rows, acc.at[dstb.at[c]], add=True)

        start_gather(0, rows0, sem0)

        @pl.loop(0, cpt // 2)
        def _(i):
            c0 = 2 * i
            c1 = c0 + 1
            start_gather(c1, rows1, sem1)
            wait_gather(c0, rows0, sem0)
            process(c0, rows0)
            start_gather(c1 + 1, rows0, sem0)
            wait_gather(c1, rows1, sem1)
            process(c1, rows1)

        wait_gather(cpt - 1, rows0, sem0)
        process(cpt - 1, rows0)

        plsc.subcore_barrier()

        @pl.when(sid < nzt)
        def _():
            pltpu.sync_copy(acc.at[pl.ds(r0, rpt)],
                            out_hbm.at[cid, pl.ds(r0, rpt)])
            if with_deg:
                pltpu.sync_copy(dacc.at[pl.ds(r0, rpt)],
                                outdeg_hbm.at[cid, pl.ds(r0, rpt)])

    return pl.kernel(
        body, out_type=out_type, mesh=mesh, scratch_types=scratch,
        compiler_params=pltpu.CompilerParams(use_tc_tiling_on_sc=False))


def _tc_layer01(p, dg, W, b, n_rows, br):
    d_in = p.shape[-1]
    d_out = W.shape[-1]

    def tc_body(p_ref, dg_ref, w_ref, b_ref, o_ref):
        h = jnp.dot(p_ref[0] + p_ref[1], w_ref[...],
                    preferred_element_type=f32)
        deg = dg_ref[0, :, 0:1] + dg_ref[1, :, 0:1]
        o_ref[...] = jnp.maximum(h + deg * b_ref[...], 0.0)

    return pl.pallas_call(
        tc_body,
        grid=(n_rows // br,),
        in_specs=[
            pl.BlockSpec((NC, br, d_in), lambda i: (0, i, 0)),
            pl.BlockSpec((NC, br, DEGW), lambda i: (0, i, 0)),
            pl.BlockSpec((d_in, d_out), lambda i: (0, 0)),
            pl.BlockSpec((1, d_out), lambda i: (0, 0)),
        ],
        out_specs=pl.BlockSpec((br, d_out), lambda i: (i, 0)),
        out_shape=jax.ShapeDtypeStruct((n_rows, d_out), f32),
    )(p, dg, W, b)


def _tc_layer2(p, dg, W1, b1, W2, b2, n_rows, br):
    d_in = p.shape[-1]
    d_mid = W1.shape[-1]
    d_out = W2.shape[-1]

    def tc_body(p_ref, dg_ref, w1_ref, b1_ref, w2_ref, b2_ref, o_ref):
        h = jnp.dot(p_ref[0] + p_ref[1], w1_ref[...],
                    preferred_element_type=f32)
        deg = dg_ref[0, :, 0:1] + dg_ref[1, :, 0:1]
        h = jnp.maximum(h + deg * b1_ref[...], 0.0)
        o_ref[...] = jnp.dot(h, w2_ref[...],
                             preferred_element_type=f32) + b2_ref[...]

    return pl.pallas_call(
        tc_body,
        grid=(n_rows // br,),
        in_specs=[
            pl.BlockSpec((NC, br, d_in), lambda i: (0, i, 0)),
            pl.BlockSpec((NC, br, DEGW), lambda i: (0, i, 0)),
            pl.BlockSpec((d_in, d_mid), lambda i: (0, 0)),
            pl.BlockSpec((1, d_mid), lambda i: (0, 0)),
            pl.BlockSpec((d_mid, d_out), lambda i: (0, 0)),
            pl.BlockSpec((1, d_out), lambda i: (0, 0)),
        ],
        out_specs=pl.BlockSpec((br, d_out), lambda i: (i, 0)),
        out_shape=jax.ShapeDtypeStruct((n_rows, d_out), f32),
    )(p, dg, W1, b1, W2, b2)


def _tc_softmax(q, n_rows, br):
    d = q.shape[-1]

    def tc_body(q_ref, o_ref):
        z = q_ref[0] + q_ref[1]
        z = z - jnp.max(z, axis=-1, keepdims=True)
        e = jnp.exp(z)
        o_ref[...] = e / jnp.sum(e, axis=-1, keepdims=True)

    return pl.pallas_call(
        tc_body,
        grid=(n_rows // br,),
        in_specs=[pl.BlockSpec((NC, br, d), lambda i: (0, i, 0))],
        out_specs=pl.BlockSpec((br, d), lambda i: (i, 0)),
        out_shape=jax.ShapeDtypeStruct((n_rows, d), f32),
    )(q)


def kernel(x, edge_index, edge_weight, W0, b0, W1, b1, W2, b2):
    n_nodes, d_feat = x.shape
    n_edges = edge_weight.shape[0]
    n_classes = W2.shape[-1]
    br = 2000

    src = edge_index[0].astype(jnp.int32)
    dst = edge_index[1].astype(jnp.int32)

    def eshape(K):
        cpt = n_edges // K // NW
        return (src.reshape(NW, cpt, K), dst.reshape(NW, cpt, K),
                edge_weight.reshape(NW, cpt, K))

    rpt = n_nodes // 10
    z128 = jnp.zeros((rpt, d_feat), f32)
    z16 = jnp.zeros((rpt, DEGW), f32)

    agg_deg = _make_agg(n_nodes, d_feat, n_edges, 16, with_deg=True)
    agg = _make_agg(n_nodes, d_feat, n_edges, 80, with_deg=False)
    agg_c = _make_agg(n_nodes, n_classes, n_edges, 80, with_deg=False)

    s16, d16, w16 = eshape(16)
    s80, d80, w80 = eshape(80)
    p0, dg = agg_deg(x, s16, d16, w16, z128, z16)
    h1 = _tc_layer01(p0, dg, W0, b0.reshape(1, -1), n_nodes, br)
    (p1,) = agg(h1, s80, d80, w80, z128, z16)
    h2 = _tc_layer2(p1, dg, W1, b1.reshape(1, -1), W2, b2.reshape(1, -1),
                    n_nodes, br)
    zc = jnp.zeros((rpt, n_classes), f32)
    (q,) = agg_c(h2, s80, d80, w80, zc, z16)
    return _tc_softmax(q, n_nodes, br)


# speedup vs baseline: 11.5271x; 1.3531x over previous
import functools

import jax
import jax.numpy as jnp
from jax import lax
from jax.experimental import pallas as pl
from jax.experimental.pallas import tpu as pltpu
from jax.experimental.pallas import tpu_sc as plsc

f32 = jnp.float32

NC = 2
NS = 16
NW = NC * NS
DEGW = 16


def _make_agg(n_nodes, d_feat, n_edges, K):
    n_chunks = n_edges // K
    cpt = n_chunks // NW
    assert cpt % 2 == 1 and cpt * NW == n_chunks
    nzt = 10
    rpt = n_nodes // nzt
    assert rpt % 8 == 0 and rpt * nzt == n_nodes
    mesh = plsc.VectorSubcoreMesh(
        core_axis_name="c", subcore_axis_name="s",
        num_cores=NC, num_subcores=NS)

    out_type = [jax.ShapeDtypeStruct((NC, n_nodes, d_feat), f32)]
    scratch = [
        pltpu.VMEM_SHARED((n_nodes, d_feat), f32),
        pltpu.VMEM((cpt, K), jnp.int32),
        pltpu.VMEM((cpt, K), jnp.int32),
        pltpu.VMEM((cpt, K), f32),
        pltpu.VMEM((K, d_feat), f32),
        pltpu.VMEM((K, d_feat), f32),
        pltpu.SemaphoreType.DMA,
        pltpu.SemaphoreType.DMA,
    ]

    def body(h_hbm, src_hbm, dst_hbm, w_hbm, z_hbm,
             out_hbm, acc, srcb, dstb, wb, rows0, rows1, sem0, sem1):
        cid = lax.axis_index("c")
        sid = lax.axis_index("s")
        tid = cid * NS + sid
        r0 = sid * rpt

        @pl.when(sid < nzt)
        def _():
            pltpu.sync_copy(z_hbm, acc.at[pl.ds(r0, rpt)])

        pltpu.sync_copy(src_hbm.at[tid], srcb)
        pltpu.sync_copy(dst_hbm.at[tid], dstb)
        pltpu.sync_copy(w_hbm.at[tid], wb)

        plsc.subcore_barrier()

        def start_gather(c, rows, sem):
            pltpu.async_copy(h_hbm.at[srcb.at[c]], rows, sem)

        def wait_gather(c, rows, sem):
            pltpu.make_async_copy(h_hbm.at[srcb.at[c]], rows, sem).wait()

        def process(c, rows):
            @plsc.parallel_loop(0, K // 16)
            def _(g):
                wvec = wb[c, pl.ds(g * 16, 16)]
                for eo in range(16):
                    e = g * 16 + eo
                    wv = jnp.full((16,), wvec[eo], f32)
                    for j in range(d_feat // 16):
                        sl = pl.ds(j * 16, 16)
                        rows[e, sl] = rows[e, sl] * wv
            pltpu.sync_copy(rows, acc.at[dstb.at[c]], add=True)

        start_gather(0, rows0, sem0)

        @pl.loop(0, cpt // 2)
        def _(i):
            c0 = 2 * i
            c1 = c0 + 1
            start_gather(c1, rows1, sem1)
            wait_gather(c0, rows0, sem0)
            process(c0, rows0)
            start_gather(c1 + 1, rows0, sem0)
            wait_gather(c1, rows1, sem1)
            process(c1, rows1)

        wait_gather(cpt - 1, rows0, sem0)
        process(cpt - 1, rows0)

        plsc.subcore_barrier()

        @pl.when(sid < nzt)
        def _():
            pltpu.sync_copy(acc.at[pl.ds(r0, rpt)],
                            out_hbm.at[cid, pl.ds(r0, rpt)])

    return pl.kernel(
        body, out_type=out_type, mesh=mesh, scratch_types=scratch,
        compiler_params=pltpu.CompilerParams(use_tc_tiling_on_sc=False))


def _make_deg(n_nodes, n_edges, K):
    n_chunks = n_edges // K
    cpt = n_chunks // NW
    assert cpt * NW == n_chunks
    nzt = 10
    rpt = n_nodes // nzt
    assert rpt % 8 == 0 and rpt * nzt == n_nodes
    mesh = plsc.VectorSubcoreMesh(
        core_axis_name="c", subcore_axis_name="s",
        num_cores=NC, num_subcores=NS)

    out_type = [jax.ShapeDtypeStruct((NC, n_nodes, DEGW), f32)]
    scratch = [
        pltpu.VMEM_SHARED((n_nodes, DEGW), f32),
        pltpu.VMEM((cpt, K), jnp.int32),
        pltpu.VMEM((cpt, K), f32),
        pltpu.VMEM((K, DEGW), f32),
    ]

    def body(dst_hbm, w_hbm, zd_hbm, out_hbm, dacc, dstb, wb, wrow):
        cid = lax.axis_index("c")
        sid = lax.axis_index("s")
        tid = cid * NS + sid
        r0 = sid * rpt

        @pl.when(sid < nzt)
        def _():
            pltpu.sync_copy(zd_hbm, dacc.at[pl.ds(r0, rpt)])

        pltpu.sync_copy(dst_hbm.at[tid], dstb)
        pltpu.sync_copy(w_hbm.at[tid], wb)

        plsc.subcore_barrier()

        @pl.loop(0, cpt)
        def _(c):
            @plsc.parallel_loop(0, K // 16)
            def _(g):
                wvec = wb[c, pl.ds(g * 16, 16)]
                for eo in range(16):
                    wrow[g * 16 + eo, :] = jnp.full((16,), wvec[eo], f32)
            pltpu.sync_copy(wrow, dacc.at[dstb.at[c]], add=True)

        plsc.subcore_barrier()

        @pl.when(sid < nzt)
        def _():
            pltpu.sync_copy(dacc.at[pl.ds(r0, rpt)],
                            out_hbm.at[cid, pl.ds(r0, rpt)])

    return pl.kernel(
        body, out_type=out_type, mesh=mesh, scratch_types=scratch,
        compiler_params=pltpu.CompilerParams(use_tc_tiling_on_sc=False))


def _tc_layer01(p, dg, W, b, n_rows, br):
    d_in = p.shape[-1]
    d_out = W.shape[-1]

    def tc_body(p_ref, dg_ref, w_ref, b_ref, o_ref):
        h = jnp.dot(p_ref[0] + p_ref[1], w_ref[...],
                    preferred_element_type=f32)
        deg = dg_ref[0, :, 0:1] + dg_ref[1, :, 0:1]
        o_ref[...] = jnp.maximum(h + deg * b_ref[...], 0.0)

    return pl.pallas_call(
        tc_body,
        grid=(n_rows // br,),
        in_specs=[
            pl.BlockSpec((NC, br, d_in), lambda i: (0, i, 0)),
            pl.BlockSpec((NC, br, DEGW), lambda i: (0, i, 0)),
            pl.BlockSpec((d_in, d_out), lambda i: (0, 0)),
            pl.BlockSpec((1, d_out), lambda i: (0, 0)),
        ],
        out_specs=pl.BlockSpec((br, d_out), lambda i: (i, 0)),
        out_shape=jax.ShapeDtypeStruct((n_rows, d_out), f32),
    )(p, dg, W, b)


def _tc_layer2(p, dg, W1, b1, W2, b2, n_rows, br):
    d_in = p.shape[-1]
    d_mid = W1.shape[-1]
    d_out = W2.shape[-1]

    def tc_body(p_ref, dg_ref, w1_ref, b1_ref, w2_ref, b2_ref, o_ref):
        h = jnp.dot(p_ref[0] + p_ref[1], w1_ref[...],
                    preferred_element_type=f32)
        deg = dg_ref[0, :, 0:1] + dg_ref[1, :, 0:1]
        h = jnp.maximum(h + deg * b1_ref[...], 0.0)
        o_ref[...] = jnp.dot(h, w2_ref[...],
                             preferred_element_type=f32) + b2_ref[...]

    return pl.pallas_call(
        tc_body,
        grid=(n_rows // br,),
        in_specs=[
            pl.BlockSpec((NC, br, d_in), lambda i: (0, i, 0)),
            pl.BlockSpec((NC, br, DEGW), lambda i: (0, i, 0)),
            pl.BlockSpec((d_in, d_mid), lambda i: (0, 0)),
            pl.BlockSpec((1, d_mid), lambda i: (0, 0)),
            pl.BlockSpec((d_mid, d_out), lambda i: (0, 0)),
            pl.BlockSpec((1, d_out), lambda i: (0, 0)),
        ],
        out_specs=pl.BlockSpec((br, d_out), lambda i: (i, 0)),
        out_shape=jax.ShapeDtypeStruct((n_rows, d_out), f32),
    )(p, dg, W1, b1, W2, b2)


def _tc_softmax(q, n_rows, br):
    d = q.shape[-1]

    def tc_body(q_ref, o_ref):
        z = q_ref[0] + q_ref[1]
        z = z - jnp.max(z, axis=-1, keepdims=True)
        e = jnp.exp(z)
        o_ref[...] = e / jnp.sum(e, axis=-1, keepdims=True)

    return pl.pallas_call(
        tc_body,
        grid=(n_rows // br,),
        in_specs=[pl.BlockSpec((NC, br, d), lambda i: (0, i, 0))],
        out_specs=pl.BlockSpec((br, d), lambda i: (i, 0)),
        out_shape=jax.ShapeDtypeStruct((n_rows, d), f32),
    )(q)


def kernel(x, edge_index, edge_weight, W0, b0, W1, b1, W2, b2):
    n_nodes, d_feat = x.shape
    n_edges = edge_weight.shape[0]
    n_classes = W2.shape[-1]
    br = 2000
    K = 80

    cpt = n_edges // K // NW
    src = edge_index[0].astype(jnp.int32).reshape(NW, cpt, K)
    dst = edge_index[1].astype(jnp.int32).reshape(NW, cpt, K)
    w3 = edge_weight.reshape(NW, cpt, K)

    rpt = n_nodes // 10
    z128 = jnp.zeros((rpt, d_feat), f32)
    z16 = jnp.zeros((rpt, DEGW), f32)

    agg = _make_agg(n_nodes, d_feat, n_edges, K)
    agg_c = _make_agg(n_nodes, n_classes, n_edges, K)
    deg_k = _make_deg(n_nodes, n_edges, K)

    (dg,) = deg_k(dst, w3, z16)
    (p0,) = agg(x, src, dst, w3, z128)
    h1 = _tc_layer01(p0, dg, W0, b0.reshape(1, -1), n_nodes, br)
    (p1,) = agg(h1, src, dst, w3, z128)
    h2 = _tc_layer2(p1, dg, W1, b1.reshape(1, -1), W2, b2.reshape(1, -1),
                    n_nodes, br)
    zc = jnp.zeros((rpt, n_classes), f32)
    (q,) = agg_c(h2, src, dst, w3, zc)
    return _tc_softmax(q, n_nodes, br)


# speedup vs baseline: 12.3488x; 1.0713x over previous
import functools

import jax
import jax.numpy as jnp
from jax import lax
from jax.experimental import pallas as pl
from jax.experimental.pallas import tpu as pltpu
from jax.experimental.pallas import tpu_sc as plsc

f32 = jnp.float32

NC = 2
NS = 16
NW = NC * NS
DEGW = 16


def _make_agg(n_nodes, d_feat, n_edges, K):
    n_chunks = n_edges // K
    cpt = n_chunks // NW
    assert cpt % 2 == 1 and cpt * NW == n_chunks
    nzt = 10
    rpt = n_nodes // nzt
    assert rpt % 8 == 0 and rpt * nzt == n_nodes
    mesh = plsc.VectorSubcoreMesh(
        core_axis_name="c", subcore_axis_name="s",
        num_cores=NC, num_subcores=NS)

    out_type = [jax.ShapeDtypeStruct((NC, n_nodes, d_feat), f32)]
    scratch = [
        pltpu.VMEM_SHARED((n_nodes, d_feat), f32),
        pltpu.VMEM((cpt, K), jnp.int32),
        pltpu.VMEM((cpt, K), jnp.int32),
        pltpu.VMEM((cpt, K), f32),
        pltpu.VMEM((K, d_feat), f32),
        pltpu.VMEM((K, d_feat), f32),
        pltpu.SemaphoreType.DMA,
        pltpu.SemaphoreType.DMA,
        pltpu.SemaphoreType.DMA,
        pltpu.SemaphoreType.DMA,
    ]

    def body(h_hbm, src_hbm, dst_hbm, w_hbm, z_hbm,
             out_hbm, acc, srcb, dstb, wb, rows0, rows1,
             sem0, sem1, ssem0, ssem1):
        cid = lax.axis_index("c")
        sid = lax.axis_index("s")
        tid = cid * NS + sid
        r0 = sid * rpt

        @pl.when(sid < nzt)
        def _():
            pltpu.sync_copy(z_hbm, acc.at[pl.ds(r0, rpt)])

        pltpu.sync_copy(src_hbm.at[tid], srcb)
        pltpu.sync_copy(dst_hbm.at[tid], dstb)
        pltpu.sync_copy(w_hbm.at[tid], wb)

        plsc.subcore_barrier()

        def start_gather(c, rows, sem):
            pltpu.async_copy(h_hbm.at[srcb.at[c]], rows, sem)

        def wait_gather(c, rows, sem):
            pltpu.make_async_copy(h_hbm.at[srcb.at[c]], rows, sem).wait()

        def scale(c, rows):
            @plsc.parallel_loop(0, K // 16)
            def _(g):
                wvec = wb[c, pl.ds(g * 16, 16)]
                for eo in range(16):
                    e = g * 16 + eo
                    wv = wvec.at[jnp.full((16,), eo, jnp.int32)].get(
                        mode="promise_in_bounds")
                    for j in range(d_feat // 16):
                        sl = pl.ds(j * 16, 16)
                        rows[e, sl] = rows[e, sl] * wv

        def process(c, rows):
            scale(c, rows)
            pltpu.sync_copy(rows, acc.at[dstb.at[c]], add=True)

        start_gather(0, rows0, sem0)

        @pl.loop(0, cpt // 2)
        def _(i):
            c0 = 2 * i
            c1 = c0 + 1
            start_gather(c1, rows1, sem1)
            wait_gather(c0, rows0, sem0)
            process(c0, rows0)
            start_gather(c1 + 1, rows0, sem0)
            wait_gather(c1, rows1, sem1)
            process(c1, rows1)

        wait_gather(cpt - 1, rows0, sem0)
        process(cpt - 1, rows0)

        plsc.subcore_barrier()

        @pl.when(sid < nzt)
        def _():
            pltpu.sync_copy(acc.at[pl.ds(r0, rpt)],
                            out_hbm.at[cid, pl.ds(r0, rpt)])

    return pl.kernel(
        body, out_type=out_type, mesh=mesh, scratch_types=scratch,
        compiler_params=pltpu.CompilerParams(use_tc_tiling_on_sc=False))


def _tc_affine(x, W, b, n_rows, br):
    d_in = x.shape[-1]
    d_out = W.shape[-1]

    def tc_body(x_ref, w_ref, b_ref, o_ref):
        o_ref[...] = jnp.dot(x_ref[...], w_ref[...],
                             preferred_element_type=f32) + b_ref[...]

    return pl.pallas_call(
        tc_body,
        grid=(n_rows // br,),
        in_specs=[
            pl.BlockSpec((br, d_in), lambda i: (i, 0)),
            pl.BlockSpec((d_in, d_out), lambda i: (0, 0)),
            pl.BlockSpec((1, d_out), lambda i: (0, 0)),
        ],
        out_specs=pl.BlockSpec((br, d_out), lambda i: (i, 0)),
        out_shape=jax.ShapeDtypeStruct((n_rows, d_out), f32),
    )(x, W, b)


def _tc_merge_relu_affine(p, W, b, n_rows, br):
    d_in = p.shape[-1]
    d_out = W.shape[-1]

    def tc_body(p_ref, w_ref, b_ref, o_ref):
        h = jnp.maximum(p_ref[0] + p_ref[1], 0.0)
        o_ref[...] = jnp.dot(h, w_ref[...],
                             preferred_element_type=f32) + b_ref[...]

    return pl.pallas_call(
        tc_body,
        grid=(n_rows // br,),
        in_specs=[
            pl.BlockSpec((NC, br, d_in), lambda i: (0, i, 0)),
            pl.BlockSpec((d_in, d_out), lambda i: (0, 0)),
            pl.BlockSpec((1, d_out), lambda i: (0, 0)),
        ],
        out_specs=pl.BlockSpec((br, d_out), lambda i: (i, 0)),
        out_shape=jax.ShapeDtypeStruct((n_rows, d_out), f32),
    )(p, W, b)


def _tc_softmax(q, n_rows, br):
    d = q.shape[-1]

    def tc_body(q_ref, o_ref):
        z = q_ref[0] + q_ref[1]
        z = z - jnp.max(z, axis=-1, keepdims=True)
        e = jnp.exp(z)
        o_ref[...] = e / jnp.sum(e, axis=-1, keepdims=True)

    return pl.pallas_call(
        tc_body,
        grid=(n_rows // br,),
        in_specs=[pl.BlockSpec((NC, br, d), lambda i: (0, i, 0))],
        out_specs=pl.BlockSpec((br, d), lambda i: (i, 0)),
        out_shape=jax.ShapeDtypeStruct((n_rows, d), f32),
    )(q)


def kernel(x, edge_index, edge_weight, W0, b0, W1, b1, W2, b2):
    n_nodes, d_feat = x.shape
    n_edges = edge_weight.shape[0]
    n_classes = W2.shape[-1]
    br = 2000
    K = 80

    cpt = n_edges // K // NW
    src = edge_index[0].astype(jnp.int32).reshape(NW, cpt, K)
    dst = edge_index[1].astype(jnp.int32).reshape(NW, cpt, K)
    w3 = edge_weight.reshape(NW, cpt, K)

    rpt = n_nodes // 10
    z128 = jnp.zeros((rpt, d_feat), f32)
    zc = jnp.zeros((rpt, n_classes), f32)

    agg = _make_agg(n_nodes, d_feat, n_edges, K)
    agg_c = _make_agg(n_nodes, n_classes, n_edges, K)

    t0 = _tc_affine(x, W0, b0.reshape(1, -1), n_nodes, br)
    (a0,) = agg(t0, src, dst, w3, z128)
    t1 = _tc_merge_relu_affine(a0, W1, b1.reshape(1, -1), n_nodes, br)
    (a1,) = agg(t1, src, dst, w3, z128)
    t2 = _tc_merge_relu_affine(a1, W2, b2.reshape(1, -1), n_nodes, br)
    (q,) = agg_c(t2, src, dst, w3, zc)
    return _tc_softmax(q, n_nodes, br)
